# Initial kernel scaffold; baseline (speedup 1.0000x reference)
import jax
import jax.numpy as jnp
from jax.experimental import pallas as pl


def kernel(X, neigh_idx, W):
    raise NotImplementedError("write your pallas kernel here")


# speedup vs baseline: 1.1991x; 1.1991x over previous
import functools

import jax
import jax.numpy as jnp
from jax import lax
from jax.experimental import pallas as pl
from jax.experimental.pallas import tpu as pltpu
from jax.experimental.pallas import tpu_sc as plsc

N_NODES = 10000
K = 32
D = 128

NC = 2
NS = 16
NW = NC * NS

NPAD = 10240
PER_W = NPAD // NW
CHUNK = 8
N_CHUNKS = PER_W // CHUNK


def _mm_body(x_ref, w_ref, z_ref):
    z_ref[...] = lax.dot_general(
        x_ref[...], w_ref[...],
        dimension_numbers=(((1,), (1,)), ((), ())),
        preferred_element_type=jnp.float32,
    )


def _matmul(x_pad, w):
    blk = 512
    grid = NPAD // blk
    return pl.pallas_call(
        _mm_body,
        grid=(grid,),
        in_specs=[
            pl.BlockSpec((blk, D), lambda i: (i, 0)),
            pl.BlockSpec((D, D), lambda i: (0, 0)),
        ],
        out_specs=pl.BlockSpec((blk, D), lambda i: (i, 0)),
        out_shape=jax.ShapeDtypeStruct((NPAD, D), jnp.float32),
    )(x_pad, w)


def _sc_body(z_hbm, nidx_hbm, out_hbm, idx_v, gat0, gat1, own_v, out_v, sem):
    wid = lax.axis_index("s") * NC + lax.axis_index("c")
    node_base = wid * PER_W
    pltpu.sync_copy(nidx_hbm.at[pl.ds(wid * (PER_W * K // D), PER_W * K // D)],
                    idx_v)

    def chunk_body(t, _):
        base = node_base + t * CHUNK

        cp0 = pltpu.async_copy(z_hbm.at[idx_v.at[2 * t]], gat0, sem)
        cp1 = pltpu.async_copy(z_hbm.at[idx_v.at[2 * t + 1]], gat1, sem)
        pltpu.sync_copy(z_hbm.at[pl.ds(base, CHUNK)], own_v)
        cp0.wait()
        cp1.wait()

        for h, gat in ((0, gat0), (1, gat1)):
            for n in range(4):
                node = h * 4 + n
                for d in range(D // 16):
                    sl = pl.ds(d * 16, 16)
                    acc = gat[n * K, sl]
                    for j in range(1, K):
                        acc = acc + gat[n * K + j, sl]
                    val = own_v[node, sl] - acc * (1.0 / K)
                    out_v[node, sl] = jnp.maximum(val, 0.0)

        pltpu.sync_copy(out_v, out_hbm.at[pl.ds(base, CHUNK)])
        return _

    lax.fori_loop(0, N_CHUNKS, chunk_body, None)


_sc_call = functools.partial(
    pl.kernel,
    out_type=jax.ShapeDtypeStruct((NPAD, D), jnp.float32),
    mesh=plsc.VectorSubcoreMesh(core_axis_name="c", subcore_axis_name="s"),
    scratch_types=[
        pltpu.VMEM((PER_W * K // D, D), jnp.int32),
        pltpu.VMEM((D, D), jnp.float32),
        pltpu.VMEM((D, D), jnp.float32),
        pltpu.VMEM((CHUNK, D), jnp.float32),
        pltpu.VMEM((CHUNK, D), jnp.float32),
        pltpu.SemaphoreType.DMA,
    ],
)(_sc_body)


def kernel(X, neigh_idx, W):
    x_pad = jnp.zeros((NPAD, D), jnp.float32).at[:N_NODES].set(X)
    nidx_pad = jnp.zeros((NPAD, K), jnp.int32).at[:N_NODES].set(neigh_idx)
    nidx2d = nidx_pad.reshape(NPAD * K // D, D)
    z = _matmul(x_pad, W)
    out = _sc_call(z, nidx2d)
    return out[:N_NODES]


# speedup vs baseline: 1.4134x; 1.1788x over previous
import functools

import jax
import jax.numpy as jnp
from jax import lax
from jax.experimental import pallas as pl
from jax.experimental.pallas import tpu as pltpu
from jax.experimental.pallas import tpu_sc as plsc

N_NODES = 10000
K = 32
D = 128

NC = 2
NS = 16
NW = NC * NS

NPAD = 10240
PER_W = NPAD // NW
CHUNK = 8
N_CHUNKS = PER_W // CHUNK
IDX_ROWS = PER_W * K // D


def _mm_body(x_ref, w_ref, z_ref):
    z_ref[...] = lax.dot_general(
        x_ref[...], w_ref[...],
        dimension_numbers=(((1,), (1,)), ((), ())),
        preferred_element_type=jnp.float32,
    )


def _matmul(x_pad, w):
    blk = 512
    grid = NPAD // blk
    return pl.pallas_call(
        _mm_body,
        grid=(grid,),
        in_specs=[
            pl.BlockSpec((blk, D), lambda i: (i, 0)),
            pl.BlockSpec((D, D), lambda i: (0, 0)),
        ],
        out_specs=pl.BlockSpec((blk, D), lambda i: (i, 0)),
        out_shape=jax.ShapeDtypeStruct((NPAD, D), jnp.float32),
    )(x_pad, w)


def _reduce_chunk(gat0, gat1, own_all, out_v, t):
    for h, gat in ((0, gat0), (1, gat1)):
        for n in range(4):
            node = h * 4 + n
            for d in range(D // 16):
                sl = pl.ds(d * 16, 16)
                acc = gat[n * K, sl]
                for j in range(1, K):
                    acc = acc + gat[n * K + j, sl]
                val = own_all[t * CHUNK + node, sl] - acc * (1.0 / K)
                out_v[node, sl] = jnp.maximum(val, 0.0)


def _sc_body(z_hbm, nidx_hbm, out_hbm,
             idx_v, ga0, ga1, gb0, gb1, own_all, out_a, out_b,
             sem_a, sem_b, sem_oa, sem_ob, sem_own):
    wid = lax.axis_index("s") * NC + lax.axis_index("c")
    node_base = wid * PER_W

    own_cp = pltpu.async_copy(z_hbm.at[pl.ds(node_base, PER_W)], own_all,
                              sem_own)
    pltpu.sync_copy(nidx_hbm.at[pl.ds(wid * IDX_ROWS, IDX_ROWS)], idx_v)

    def gather(t, g0, g1, sem):
        tc = jnp.minimum(t, N_CHUNKS - 1)
        pltpu.async_copy(z_hbm.at[idx_v.at[2 * tc]], g0, sem)
        pltpu.async_copy(z_hbm.at[idx_v.at[2 * tc + 1]], g1, sem)

    def wait_pair(g0, g1, sem):
        pltpu.make_async_copy(z_hbm.at[pl.ds(0, D)], g0, sem).wait()
        pltpu.make_async_copy(z_hbm.at[pl.ds(0, D)], g1, sem).wait()

    def store_out(t, out_v, sem):
        base = node_base + t * CHUNK
        pltpu.async_copy(out_v, out_hbm.at[pl.ds(base, CHUNK)], sem)

    def wait_store(out_v, sem):
        pltpu.make_async_copy(out_v, out_hbm.at[pl.ds(0, CHUNK)], sem).wait()

    gather(0, ga0, ga1, sem_a)
    own_cp.wait()

    def loop_body(s, _):
        t0 = 2 * s
        t1 = 2 * s + 1
        gather(t1, gb0, gb1, sem_b)
        wait_pair(ga0, ga1, sem_a)
        _reduce_chunk(ga0, ga1, own_all, out_a, t0)
        store_out(t0, out_a, sem_oa)
        gather(t1 + 1, ga0, ga1, sem_a)
        wait_pair(gb0, gb1, sem_b)
        _reduce_chunk(gb0, gb1, own_all, out_b, t1)
        store_out(t1, out_b, sem_ob)
        wait_store(out_a, sem_oa)
        wait_store(out_b, sem_ob)
        return _

    lax.fori_loop(0, N_CHUNKS // 2, loop_body, None)
    wait_pair(ga0, ga1, sem_a)


_sc_call = functools.partial(
    pl.kernel,
    out_type=jax.ShapeDtypeStruct((NPAD, D), jnp.float32),
    mesh=plsc.VectorSubcoreMesh(core_axis_name="c", subcore_axis_name="s"),
    scratch_types=[
        pltpu.VMEM((IDX_ROWS, D), jnp.int32),
        pltpu.VMEM((D, D), jnp.float32),
        pltpu.VMEM((D, D), jnp.float32),
        pltpu.VMEM((D, D), jnp.float32),
        pltpu.VMEM((D, D), jnp.float32),
        pltpu.VMEM((PER_W, D), jnp.float32),
        pltpu.VMEM((CHUNK, D), jnp.float32),
        pltpu.VMEM((CHUNK, D), jnp.float32),
        pltpu.SemaphoreType.DMA,
        pltpu.SemaphoreType.DMA,
        pltpu.SemaphoreType.DMA,
        pltpu.SemaphoreType.DMA,
        pltpu.SemaphoreType.DMA,
    ],
)(_sc_body)


def kernel(X, neigh_idx, W):
    x_pad = jnp.zeros((NPAD, D), jnp.float32).at[:N_NODES].set(X)
    nidx_pad = jnp.zeros((NPAD, K), jnp.int32).at[:N_NODES].set(neigh_idx)
    nidx2d = nidx_pad.reshape(NPAD * K // D, D)
    z = _matmul(x_pad, W)
    out = _sc_call(z, nidx2d)
    return out[:N_NODES]
